# baseline (speedup 1.0000x reference)
from functools import partial

import jax
import jax.numpy as jnp
from jax import lax
from jax.experimental import pallas as pl
from jax.experimental.pallas import tpu as pltpu
from jax.experimental.pallas import tpu_sc as plsc

_WINDOW = 128
_NUM_CORES = 2
_NUM_SUBCORES = 16
_NW = _NUM_CORES * _NUM_SUBCORES
_N = 100000
_BPW = 3128
_BPW_LAST = _N - (_NW - 1) * _BPW
_TAIL = _BPW - 24 * _WINDOW
_TAIL_LAST = _BPW_LAST - 23 * _WINDOW
_H1W = 12
_H1 = _H1W * _WINDOW
_H2 = _BPW - _H1
_H2_LAST = _BPW_LAST - _H1


def _gather_fn(embed_dim: int):
    mesh = plsc.VectorSubcoreMesh(core_axis_name="core", subcore_axis_name="subcore")

    @partial(
        pl.kernel,
        out_type=jax.ShapeDtypeStruct((_N, embed_dim), jnp.float32),
        mesh=mesh,
        scratch_types=[
            pltpu.VMEM((_BPW,), jnp.int32),
            pltpu.VMEM((_BPW, embed_dim), jnp.float32),
            pltpu.SemaphoreType.DMA,
            pltpu.SemaphoreType.DMA,
            pltpu.SemaphoreType.DMA,
            pltpu.SemaphoreType.DMA,
        ],
        compiler_params=pltpu.CompilerParams(use_tc_tiling_on_sc=False),
    )
    def gather(w_hbm, i_hbm, o_hbm, idx_v, rows_v, sem_i, sem_g, sem_g2, sem_w):
        wid = lax.axis_index("subcore") * _NUM_CORES + lax.axis_index("core")
        base = wid * _BPW
        is_last = wid == _NW - 1

        @pl.when(~is_last)
        def _():
            pltpu.async_copy(i_hbm.at[pl.ds(base, _BPW)], idx_v, sem_i).wait()

        @pl.when(is_last)
        def _():
            pltpu.async_copy(
                i_hbm.at[pl.ds(base, _BPW_LAST)],
                idx_v.at[pl.ds(0, _BPW_LAST)],
                sem_i,
            ).wait()

        nfull = lax.select(is_last, 23, 24)

        @pl.loop(0, _H1W)
        def _(j):
            pltpu.async_copy(
                w_hbm.at[idx_v.at[pl.ds(j * _WINDOW, _WINDOW)]],
                rows_v.at[pl.ds(j * _WINDOW, _WINDOW)],
                sem_g,
            )

        @pl.loop(_H1W, nfull)
        def _(j):
            pltpu.async_copy(
                w_hbm.at[idx_v.at[pl.ds(j * _WINDOW, _WINDOW)]],
                rows_v.at[pl.ds(j * _WINDOW, _WINDOW)],
                sem_g2,
            )

        @pl.when(~is_last)
        def _():
            pltpu.async_copy(
                w_hbm.at[idx_v.at[pl.ds(24 * _WINDOW, _TAIL)]],
                rows_v.at[pl.ds(24 * _WINDOW, _TAIL)],
                sem_g2,
            )

        @pl.when(is_last)
        def _():
            pltpu.async_copy(
                w_hbm.at[idx_v.at[pl.ds(23 * _WINDOW, _TAIL_LAST)]],
                rows_v.at[pl.ds(23 * _WINDOW, _TAIL_LAST)],
                sem_g2,
            )

        pltpu.make_async_copy(
            o_hbm.at[pl.ds(base, _H1)], rows_v.at[pl.ds(0, _H1)], sem_g
        ).wait()
        pltpu.async_copy(
            rows_v.at[pl.ds(0, _H1)], o_hbm.at[pl.ds(base, _H1)], sem_w
        )

        @pl.when(~is_last)
        def _():
            pltpu.make_async_copy(
                o_hbm.at[pl.ds(base + _H1, _H2)],
                rows_v.at[pl.ds(_H1, _H2)],
                sem_g2,
            ).wait()
            pltpu.sync_copy(
                rows_v.at[pl.ds(_H1, _H2)], o_hbm.at[pl.ds(base + _H1, _H2)]
            )

        @pl.when(is_last)
        def _():
            pltpu.make_async_copy(
                o_hbm.at[pl.ds(base + _H1, _H2_LAST)],
                rows_v.at[pl.ds(_H1, _H2_LAST)],
                sem_g2,
            ).wait()
            pltpu.sync_copy(
                rows_v.at[pl.ds(_H1, _H2_LAST)],
                o_hbm.at[pl.ds(base + _H1, _H2_LAST)],
            )

        pltpu.make_async_copy(
            rows_v.at[pl.ds(0, _H1)], o_hbm.at[pl.ds(base, _H1)], sem_w
        ).wait()

    return gather


def kernel(node_type, weight):
    idx = node_type.reshape(-1)
    return _gather_fn(weight.shape[1])(weight, idx)


# speedup vs baseline: 1.0210x; 1.0210x over previous
from functools import partial

import jax
import jax.numpy as jnp
from jax import lax
from jax.experimental import pallas as pl
from jax.experimental.pallas import tpu as pltpu
from jax.experimental.pallas import tpu_sc as plsc

_WINDOW = 128
_NUM_CORES = 2
_NUM_SUBCORES = 16
_NW = _NUM_CORES * _NUM_SUBCORES
_N = 100000
_BPW = 3128
_BPW_LAST = _N - (_NW - 1) * _BPW
_TAIL = _BPW - 24 * _WINDOW
_TAIL_LAST = _BPW_LAST - 23 * _WINDOW


def _gather_fn(embed_dim: int):
    mesh = plsc.VectorSubcoreMesh(core_axis_name="core", subcore_axis_name="subcore")

    @partial(
        pl.kernel,
        out_type=jax.ShapeDtypeStruct((_N, embed_dim), jnp.float32),
        mesh=mesh,
        scratch_types=[
            pltpu.VMEM((_BPW,), jnp.int32),
            pltpu.VMEM((_BPW, embed_dim), jnp.float32),
            pltpu.SemaphoreType.DMA,
            pltpu.SemaphoreType.DMA,
        ],
        compiler_params=pltpu.CompilerParams(use_tc_tiling_on_sc=False),
    )
    def gather(w_hbm, i_hbm, o_hbm, idx_v, rows_v, sem_i, sem_g):
        wid = lax.axis_index("subcore") * _NUM_CORES + lax.axis_index("core")
        base = wid * _BPW
        is_last = wid == _NW - 1

        @pl.when(~is_last)
        def _():
            pltpu.async_copy(i_hbm.at[pl.ds(base, _BPW)], idx_v, sem_i).wait()

        @pl.when(is_last)
        def _():
            pltpu.async_copy(
                i_hbm.at[pl.ds(base, _BPW_LAST)],
                idx_v.at[pl.ds(0, _BPW_LAST)],
                sem_i,
            ).wait()

        nfull = lax.select(is_last, 23, 24)

        @pl.loop(0, nfull)
        def _(j):
            pltpu.async_copy(
                w_hbm.at[idx_v.at[pl.ds(j * _WINDOW, _WINDOW)]],
                rows_v.at[pl.ds(j * _WINDOW, _WINDOW)],
                sem_g,
            )

        @pl.when(~is_last)
        def _():
            pltpu.async_copy(
                w_hbm.at[idx_v.at[pl.ds(24 * _WINDOW, _TAIL)]],
                rows_v.at[pl.ds(24 * _WINDOW, _TAIL)],
                sem_g,
            )
            pltpu.make_async_copy(o_hbm.at[pl.ds(base, _BPW)], rows_v, sem_g).wait()
            pltpu.sync_copy(rows_v, o_hbm.at[pl.ds(base, _BPW)])

        @pl.when(is_last)
        def _():
            pltpu.async_copy(
                w_hbm.at[idx_v.at[pl.ds(23 * _WINDOW, _TAIL_LAST)]],
                rows_v.at[pl.ds(23 * _WINDOW, _TAIL_LAST)],
                sem_g,
            )
            pltpu.make_async_copy(
                o_hbm.at[pl.ds(base, _BPW_LAST)],
                rows_v.at[pl.ds(0, _BPW_LAST)],
                sem_g,
            ).wait()
            pltpu.sync_copy(
                rows_v.at[pl.ds(0, _BPW_LAST)], o_hbm.at[pl.ds(base, _BPW_LAST)]
            )

    return gather


def kernel(node_type, weight):
    idx = node_type.reshape(-1)
    return _gather_fn(weight.shape[1])(weight, idx)
